# baseline (speedup 1.0000x reference)
import jax
import jax.numpy as jnp
from jax.experimental import pallas as pl
from jax.experimental.pallas import tpu as pltpu

_MASK_VALUE = -10000000.0


def _pooler_kernel(t_ref, k_ref, u_ref, mb_ref, wd_ref, bd_ref, wm_ref,
                   bm_ref, out_ref, attn_ref):
    BU = t_ref.shape[0]
    I = t_ref.shape[1]
    C = t_ref.shape[2]
    E = u_ref.shape[2]
    CH = 32
    w1 = wd_ref[:, :C]
    w2 = wd_ref[:, C:]
    b = bd_ref[0, 0]
    for u in range(BU):
        k = k_ref[u]
        s_kb = jax.lax.dot_general(w2, k, (((1,), (1,)), ((), ())),
                                   preferred_element_type=jnp.float32) + b
        m01 = mb_ref[u]
        u_part = jnp.dot(u_ref[u], wm_ref[:E, :],
                         preferred_element_type=jnp.float32) + bm_ref[:]
        for ci in range(I // CH):
            lo = ci * CH
            t = t_ref[u, lo:lo + CH, :]
            s_t = jax.lax.dot_general(t, w1, (((1,), (1,)), ((), ())),
                                      preferred_element_type=jnp.float32)
            e = jnp.exp(jnp.tanh(s_t + s_kb)) * m01
            s = jnp.sum(e, axis=1, keepdims=True)
            attn = e / s
            attn_ref[u, lo:lo + CH, :] = attn
            pooled = jnp.dot(attn, k, preferred_element_type=jnp.float32)
            h = jnp.dot(pooled, wm_ref[E:, :],
                        preferred_element_type=jnp.float32)
            out_ref[u, lo:lo + CH, :] = jnp.maximum(h + u_part, 0.0)


def kernel(target_items_context, interacted_items_context, user_embeds,
           attention_mask, w_dense, b_dense, W_mlp, b_mlp):
    U, I, C = target_items_context.shape
    J = interacted_items_context.shape[1]
    E = user_embeds.shape[1]
    BU = 8
    mask01 = attention_mask.astype(jnp.float32).reshape(U, 1, J)
    users3 = user_embeds.reshape(U, 1, E)
    wd = w_dense.reshape(1, 2 * C)
    bd = b_dense.reshape(1, 1)
    bm = b_mlp.reshape(1, C)
    out, attn = pl.pallas_call(
        _pooler_kernel,
        grid=(U // BU,),
        in_specs=[
            pl.BlockSpec((BU, I, C), lambda u: (u, 0, 0)),
            pl.BlockSpec((BU, J, C), lambda u: (u, 0, 0)),
            pl.BlockSpec((BU, 1, E), lambda u: (u, 0, 0)),
            pl.BlockSpec((BU, 1, J), lambda u: (u, 0, 0)),
            pl.BlockSpec((1, 2 * C), lambda u: (0, 0)),
            pl.BlockSpec((1, 1), lambda u: (0, 0)),
            pl.BlockSpec((E + C, C), lambda u: (0, 0)),
            pl.BlockSpec((1, C), lambda u: (0, 0)),
        ],
        out_specs=[
            pl.BlockSpec((BU, I, C), lambda u: (u, 0, 0)),
            pl.BlockSpec((BU, I, J), lambda u: (u, 0, 0)),
        ],
        out_shape=[
            jax.ShapeDtypeStruct((U, I, C), jnp.float32),
            jax.ShapeDtypeStruct((U, I, J), jnp.float32),
        ],
        compiler_params=pltpu.CompilerParams(
            dimension_semantics=("parallel",),
        ),
    )(target_items_context, interacted_items_context, users3,
      mask01, wd, bd, W_mlp, bm)
    return out, attn


# speedup vs baseline: 2.3618x; 2.3618x over previous
import jax
import jax.numpy as jnp
from jax.experimental import pallas as pl
from jax.experimental.pallas import tpu as pltpu

_MASK_VALUE = -10000000.0


def _pooler_kernel(t_ref, k_ref, u_ref, mb_ref, wd_ref, bd_ref, wm_ref,
                   bm_ref, out_ref, attn_ref):
    BU = t_ref.shape[0]
    C = t_ref.shape[2]
    E = u_ref.shape[2]
    w1 = wd_ref[:, :C]
    w2 = wd_ref[:, C:]
    b = bd_ref[0, 0]
    for u in range(BU):
        t = t_ref[u]
        k = k_ref[u]
        s_t = jax.lax.dot_general(t, w1, (((1,), (1,)), ((), ())),
                                  preferred_element_type=jnp.float32)
        s_kb = jax.lax.dot_general(w2, k, (((1,), (1,)), ((), ())),
                                   preferred_element_type=jnp.float32) + b
        e = jnp.exp(jnp.tanh(s_t + s_kb)) * mb_ref[u]
        ones_j = jnp.ones((e.shape[1], 1), jnp.float32)
        s = jnp.dot(e, ones_j, preferred_element_type=jnp.float32)
        pooled_u = jnp.dot(e, k, preferred_element_type=jnp.float32)
        r = 1.0 / s
        attn_ref[u] = e * r
        u_part = jnp.dot(u_ref[u], wm_ref[:E, :],
                         preferred_element_type=jnp.float32)
        h = jnp.dot(pooled_u * r, wm_ref[E:, :],
                    preferred_element_type=jnp.float32)
        out_ref[u] = jnp.maximum(h + u_part + bm_ref[:], 0.0)


def kernel(target_items_context, interacted_items_context, user_embeds,
           attention_mask, w_dense, b_dense, W_mlp, b_mlp):
    U, I, C = target_items_context.shape
    J = interacted_items_context.shape[1]
    E = user_embeds.shape[1]
    BU = 8
    mask01 = attention_mask.astype(jnp.float32).reshape(U, 1, J)
    users3 = user_embeds.reshape(U, 1, E)
    wd = w_dense.reshape(1, 2 * C)
    bd = b_dense.reshape(1, 1)
    bm = b_mlp.reshape(1, C)
    out, attn = pl.pallas_call(
        _pooler_kernel,
        grid=(U // BU,),
        in_specs=[
            pl.BlockSpec((BU, I, C), lambda u: (u, 0, 0)),
            pl.BlockSpec((BU, J, C), lambda u: (u, 0, 0)),
            pl.BlockSpec((BU, 1, E), lambda u: (u, 0, 0)),
            pl.BlockSpec((BU, 1, J), lambda u: (u, 0, 0)),
            pl.BlockSpec((1, 2 * C), lambda u: (0, 0)),
            pl.BlockSpec((1, 1), lambda u: (0, 0)),
            pl.BlockSpec((E + C, C), lambda u: (0, 0)),
            pl.BlockSpec((1, C), lambda u: (0, 0)),
        ],
        out_specs=[
            pl.BlockSpec((BU, I, C), lambda u: (u, 0, 0)),
            pl.BlockSpec((BU, I, J), lambda u: (u, 0, 0)),
        ],
        out_shape=[
            jax.ShapeDtypeStruct((U, I, C), jnp.float32),
            jax.ShapeDtypeStruct((U, I, J), jnp.float32),
        ],
        compiler_params=pltpu.CompilerParams(
            dimension_semantics=("parallel",),
        ),
    )(target_items_context, interacted_items_context, users3,
      mask01, wd, bd, W_mlp, bm)
    return out, attn
